# Initial kernel scaffold; baseline (speedup 1.0000x reference)
import jax
import jax.numpy as jnp
from jax.experimental import pallas as pl


def kernel(x_type, x_feat, edge_index, edge_attr, batch, node_emb, W_e, b_e, W_c, bn_g, bn_b, W_o1, b_o1, o_g, o_b, W_o2, b_o2):
    raise NotImplementedError("write your pallas kernel here")


# speedup vs baseline: 3.0142x; 3.0142x over previous
import functools

import jax
import jax.numpy as jnp
from jax import lax
from jax.experimental import pallas as pl
from jax.experimental.pallas import tpu as pltpu
import jax.experimental.pallas.tpu_sc as plsc

N = 10000
E = 320000
NHID = 128
NLAYER = 3
D_EDGE = 16
EMB = 15
NTYPES = 100
NGRAPH = 16
NOUT = 1

NC = 2
NS = 16
NW = NC * NS
EPT = E // NW
C = 80
NCHUNK = EPT // C
NPAD = 10240
RPT = NPAD // NS
F32 = jnp.float32


def _encode_body(xt_ref, feat_ref, emb_ref, o_ref):
    xt = xt_ref[...]
    oh = (xt == lax.broadcasted_iota(jnp.int32, (N, NTYPES), 1)).astype(F32)
    o_ref[...] = jnp.dot(oh, emb_ref[...],
                         preferred_element_type=F32,
                 precision=lax.Precision.HIGHEST) + feat_ref[...]


def _encode(x_type2, feat_pad, emb_pad):
    return pl.pallas_call(
        _encode_body,
        out_shape=jax.ShapeDtypeStruct((N, NHID), F32),
    )(x_type2, feat_pad, emb_pad)


EBLK = 2000


def _edge_mlp_body(a_ref, w_ref, b_ref, o_ref):
    h = jnp.dot(a_ref[...], w_ref[...], preferred_element_type=F32)
    o_ref[...] = jnp.maximum(h + b_ref[...], 0.0)


def _edge_mlp(attr8, w_bd, b_tile):
    g = (E // 8) // EBLK
    return pl.pallas_call(
        _edge_mlp_body,
        grid=(g,),
        in_specs=[
            pl.BlockSpec((EBLK, 8 * D_EDGE), lambda i: (i, 0)),
            pl.BlockSpec((8 * D_EDGE, 8 * NHID), lambda i: (0, 0)),
            pl.BlockSpec((1, 8 * NHID), lambda i: (0, 0)),
        ],
        out_specs=pl.BlockSpec((EBLK, 8 * NHID), lambda i: (i, 0)),
        out_shape=jax.ShapeDtypeStruct((E // 8, 8 * NHID), F32),
    )(attr8, w_bd, b_tile)


def _conv_mm_body(x_ref, agg_ref, w_ref, o_ref):
    s = x_ref[...] + agg_ref[0, :N] + agg_ref[1, :N]
    o_ref[...] = jnp.dot(s, w_ref[...], preferred_element_type=F32)


def _conv_mm(x, agg2, w):
    return pl.pallas_call(
        _conv_mm_body,
        out_shape=jax.ShapeDtypeStruct((N, NHID), F32),
    )(x, agg2, w)


def _head_body(x_ref, batch_ref, w1_ref, b1_ref, g_ref, b_ref, w2_ref, b2_ref,
               o_ref):
    bt = batch_ref[...]
    oh = (bt == lax.broadcasted_iota(jnp.int32, (NGRAPH, N), 0)).astype(F32)
    pooled = jnp.dot(oh, x_ref[...], preferred_element_type=F32,
                 precision=lax.Precision.HIGHEST)
    h = jnp.dot(pooled, w1_ref[...], preferred_element_type=F32) + b1_ref[...]
    mu = jnp.mean(h, axis=0, keepdims=True)
    var = jnp.mean(jnp.square(h - mu), axis=0, keepdims=True)
    h = (h - mu) / jnp.sqrt(var + 1e-5) * g_ref[...] + b_ref[...]
    h = jnp.maximum(h, 0.0)
    o_ref[...] = jnp.dot(h, w2_ref[...], preferred_element_type=F32) + b2_ref[...]


def _head(x, batch2, w1, b1, g, b, w2_pad, b2_pad):
    return pl.pallas_call(
        _head_body,
        out_shape=jax.ShapeDtypeStruct((NGRAPH, NHID), F32),
    )(x, batch2, w1, b1, g, b, w2_pad, b2_pad)


@functools.lru_cache(maxsize=None)
def _make_aggregate(with_ew: bool):
    mesh = plsc.VectorSubcoreMesh(core_axis_name="c", subcore_axis_name="s",
                                  num_cores=NC, num_subcores=NS)

    scratch = [
        pltpu.VMEM((NCHUNK, C), jnp.int32),
        pltpu.VMEM((NCHUNK, C), jnp.int32),
        pltpu.VMEM((C, NHID), F32),
        pltpu.VMEM_SHARED((NPAD, NHID), F32),
        pltpu.SemaphoreType.DMA,
    ]
    if with_ew:
        scratch.insert(2, pltpu.VMEM((C,), F32))

    def body(*refs):
        if with_ew:
            (e_hbm, x_hbm, src3, dst3, ewf, out_hbm,
             src_v, dst_v, ew_c, m_buf, acc, sem) = refs
        else:
            (e_hbm, x_hbm, src3, dst3, out_hbm,
             src_v, dst_v, m_buf, acc, sem) = refs
        cid = lax.axis_index("c")
        sid = lax.axis_index("s")
        wid = cid * NS + sid

        pltpu.sync_copy(src3.at[wid], src_v)
        pltpu.sync_copy(dst3.at[wid], dst_v)

        zv = jnp.zeros((16,), F32)

        def zrow(i, _):
            for k in range(NHID // 16):
                m_buf[i, pl.ds(k * 16, 16)] = zv
            return 0

        lax.fori_loop(0, C, zrow, 0, unroll=False)
        row0 = sid * RPT
        for t in range(RPT // C):
            pltpu.sync_copy(m_buf, acc.at[pl.ds(row0 + t * C, C)])
        plsc.subcore_barrier()

        ebase = wid * EPT

        def chunk(j, _):
            pltpu.sync_copy(e_hbm.at[pl.ds(ebase + j * C, C)], m_buf)
            pltpu.async_copy(x_hbm.at[src_v.at[j]], m_buf, sem,
                             add=True).wait()

            if with_ew:
                pltpu.sync_copy(ewf.at[pl.ds(ebase + j * C, C)], ew_c)

                def egrp(g, _):
                    ew16 = ew_c[pl.ds(g * 16, 16)]
                    for i in range(16):
                        ews = lax.gather(
                            ew16,
                            jnp.full((16, 1), i, jnp.int32),
                            lax.GatherDimensionNumbers(
                                offset_dims=(),
                                collapsed_slice_dims=(0,),
                                start_index_map=(0,)),
                            (1,),
                            mode=lax.GatherScatterMode.PROMISE_IN_BOUNDS)
                        row = g * 16 + i
                        for k in range(NHID // 16):
                            sl = pl.ds(k * 16, 16)
                            m_buf[row, sl] = (
                                jnp.maximum(m_buf[row, sl], 0.0) * ews)
                    return 0

                lax.fori_loop(0, C // 16, egrp, 0, unroll=False)
            else:
                def erow(i, _):
                    for k in range(NHID // 16):
                        sl = pl.ds(k * 16, 16)
                        m_buf[i, sl] = jnp.maximum(m_buf[i, sl], 0.0)
                    return 0

                lax.fori_loop(0, C, erow, 0, unroll=False)

            pltpu.sync_copy(m_buf, acc.at[dst_v.at[j]], add=True)
            return 0

        lax.fori_loop(0, NCHUNK, chunk, 0, unroll=False)
        plsc.subcore_barrier()
        pltpu.sync_copy(acc.at[pl.ds(row0, RPT)],
                        out_hbm.at[cid].at[pl.ds(row0, RPT)])

    kern = pl.kernel(
        body,
        out_type=jax.ShapeDtypeStruct((NC, NPAD, NHID), F32),
        mesh=mesh,
        scratch_types=scratch,
    )
    return kern


def _aggregate_ew(*args):
    return _make_aggregate(True)(*args)


def _aggregate(*args):
    return _make_aggregate(False)(*args)


def kernel(x_type, x_feat, edge_index, edge_attr, batch, node_emb,
           W_e, b_e, W_c, bn_g, bn_b, W_o1, b_o1, o_g, o_b, W_o2, b_o2):
    x_type2 = x_type.reshape(N, 1).astype(jnp.int32)
    feat_pad = jnp.pad(x_feat, ((0, 0), (NHID - EMB, 0)))
    emb_pad = jnp.pad(node_emb, ((0, 0), (0, EMB)))

    attr8 = edge_attr.reshape(E // 8, 8 * D_EDGE)
    w_bd_l = []
    for l in range(NLAYER):
        w_bd = jnp.zeros((8 * D_EDGE, 8 * NHID), F32)
        for j in range(8):
            w_bd = w_bd.at[j * D_EDGE:(j + 1) * D_EDGE,
                           j * NHID:(j + 1) * NHID].set(W_e[l])
        w_bd_l.append(w_bd)
        del w_bd
    b_tile = jnp.tile(b_e, (1, 8)).reshape(NLAYER, 1, 8 * NHID)

    src3 = edge_index[0].reshape(NW, NCHUNK, C).astype(jnp.int32)
    dst3 = edge_index[1].reshape(NW, NCHUNK, C).astype(jnp.int32)
    ewf = edge_attr[:, 1].reshape(E)

    batch2 = batch.reshape(1, N).astype(jnp.int32)
    w2_pad = jnp.pad(W_o2, ((0, 0), (0, NHID - NOUT)))
    b2_pad = jnp.pad(b_o2, ((0, NHID - NOUT))).reshape(1, NHID)

    x = _encode(x_type2, feat_pad, emb_pad)
    for l in range(NLAYER):
        e8 = _edge_mlp(attr8, w_bd_l[l], b_tile[l])
        e_full = e8.reshape(E, NHID)
        if l == 0:
            agg2 = _aggregate_ew(e_full, x, src3, dst3, ewf)
        else:
            agg2 = _aggregate(e_full, x, src3, dst3)
        h = _conv_mm(x, agg2, W_c[l])
        mu = jnp.mean(h, axis=0, keepdims=True)
        var = jnp.var(h, axis=0, keepdims=True)
        h = (h - mu) / jnp.sqrt(var + 1e-5) * bn_g[l] + bn_b[l]
        x = jax.nn.relu(h) + x
    res = _head(x, batch2, W_o1, b_o1.reshape(1, NHID),
                o_g.reshape(1, NHID), o_b.reshape(1, NHID), w2_pad, b2_pad)
    return res[:, :NOUT]
